# baseline (speedup 1.0000x reference)
import functools

import jax
import jax.numpy as jnp
from jax import lax
from jax.experimental import pallas as pl
from jax.experimental.pallas import tpu as pltpu
from jax.experimental.pallas import tpu_sc as plsc

_L = 2048
_D = 64
_NW = 32
_RPW = _L // _NW


def _sc_body(table_hbm, out_hbm, tvx, sem):
    nc = 2
    wid = lax.axis_index("s") * nc + lax.axis_index("c")
    base = wid * _RPW

    pltpu.sync_copy(table_hbm, tvx.at[pl.ds(384, 257)])
    t0 = [tvx[384, pl.ds(16 * l, 16)] for l in range(4)]
    t256 = [tvx[640, pl.ds(16 * l, 16)] for l in range(4)]

    def _fill_lo(r, carry):
        for l in range(4):
            tvx[r, pl.ds(16 * l, 16)] = t0[l]
        return carry

    def _fill_hi(r, carry):
        for l in range(4):
            tvx[r, pl.ds(16 * l, 16)] = t256[l]
        return carry

    lax.fori_loop(0, 384, _fill_lo, 0)
    lax.fori_loop(641, 1024, _fill_hi, 0)

    def _mk_copies(i):
        wstart = (jnp.clip(i - 128, 0, 1536) // 256) * 256
        nw = wstart // 256
        copies = [
            pltpu.make_async_copy(
                tvx.at[pl.ds(wstart - i + 512, 512)],
                out_hbm.at[i, pl.ds(wstart, 512)],
                sem,
            )
        ]
        for c in range(6):
            is_pre = c < nw
            dst0 = jnp.where(is_pre, 256 * c, wstart + 512 + 256 * c - 256 * nw)
            src0 = jnp.where(is_pre, 0, 768)
            copies.append(
                pltpu.make_async_copy(
                    tvx.at[pl.ds(src0, 256)],
                    out_hbm.at[i, pl.ds(dst0, 256)],
                    sem,
                )
            )
        return copies

    def _row(r, carry):
        for cp in _mk_copies(base + r):
            cp.start()

        @pl.when(r > 0)
        def _drain_prev():
            for cp in _mk_copies(base + r - 1):
                cp.wait()

        return carry

    lax.fori_loop(0, _RPW, _row, 0)
    for cp in _mk_copies(base + _RPW - 1):
        cp.wait()


def kernel(embeddings_table, length_q, length_k):
    del length_q, length_k
    run = functools.partial(
        pl.kernel,
        out_type=jax.ShapeDtypeStruct((_L, _L, _D), jnp.float32),
        mesh=plsc.VectorSubcoreMesh(core_axis_name="c", subcore_axis_name="s"),
        scratch_types=[
            pltpu.VMEM((1024, _D), jnp.float32),
            pltpu.SemaphoreType.DMA,
        ],
    )(_sc_body)
    return run(embeddings_table)


# speedup vs baseline: 2.3763x; 2.3763x over previous
import functools

import jax
import jax.numpy as jnp
from jax import lax
from jax.experimental import pallas as pl
from jax.experimental.pallas import tpu as pltpu
from jax.experimental.pallas import tpu_sc as plsc

_L = 2048
_D = 64
_IPB = _L // 4


def _mo16(x):
    return pl.multiple_of(x, 16)


def _sc_body(tab1_hbm, out_hbm, ttmp, band1d, cfill, staged, sem):
    nc = 2
    wid = lax.axis_index("s") * nc + lax.axis_index("c")
    octet = wid % 8
    ibase = (wid // 8) * _IPB
    d8 = octet * 8
    iota16 = lax.iota(jnp.int32, 16)

    pltpu.sync_copy(tab1_hbm.at[pl.ds(d8 * 272, 8 * 272)], ttmp)
    for dl in range(8):
        tbase = dl * 272
        t0 = jnp.full((16,), ttmp[pl.ds(tbase, 16)][0], jnp.float32)
        t256 = jnp.full((16,), ttmp[pl.ds(tbase + 256, 16)][0], jnp.float32)

        def _lo(m, carry, dl=dl, t0=t0):
            band1d[pl.ds(_mo16(dl * 1024 + 16 * m), 16)] = t0
            return carry

        def _mid(m, carry, dl=dl, tbase=tbase):
            vals = plsc.load_gather(ttmp, [tbase + 1 + 16 * m + iota16])
            band1d[pl.ds(_mo16(dl * 1024 + 384 + 16 * m), 16)] = vals
            return carry

        def _hi(m, carry, dl=dl, t256=t256):
            band1d[pl.ds(_mo16(dl * 1024 + 640 + 16 * m), 16)] = t256
            return carry

        def _cf(m, carry, dl=dl, t0=t0, t256=t256):
            cfill[dl, pl.ds(_mo16(16 * m), 16)] = t0
            cfill[dl, pl.ds(_mo16(256 + 16 * m), 16)] = t256
            return carry

        lax.fori_loop(0, 24, _lo, 0)
        lax.fori_loop(0, 16, _mid, 0)
        lax.fori_loop(0, 24, _hi, 0)
        lax.fori_loop(0, 16, _cf, 0)

    def _mk_copies(i, b):
        wstart = (jnp.clip(i - 128, 0, 1536) // 256) * 256
        nw = wstart // 256
        copies = [
            pltpu.make_async_copy(
                staged.at[b],
                out_hbm.at[i, pl.ds(d8, 8), pl.ds(wstart, 512)],
                sem,
            )
        ]
        for c in range(6):
            is_pre = c < nw
            dst_j = jnp.where(is_pre, 256 * c, wstart + 512 + 256 * c - 256 * nw)
            src_j = jnp.where(is_pre, 0, 256)
            copies.append(
                pltpu.make_async_copy(
                    cfill.at[:, pl.ds(src_j, 256)],
                    out_hbm.at[i, pl.ds(d8, 8), pl.ds(dst_j, 256)],
                    sem,
                )
            )
        return copies

    def _row(r, carry):
        i = ibase + r
        b = lax.rem(r, 2)
        wstart = (jnp.clip(i - 128, 0, 1536) // 256) * 256
        obase = wstart + 511 - i

        def _stage(m, carry):
            col = obase + 16 * m + iota16
            for dl in range(8):
                vals = plsc.load_gather(band1d, [col + 1024 * dl])
                staged[b, dl, pl.ds(_mo16(16 * m), 16)] = vals
            return carry

        lax.fori_loop(0, 32, _stage, 0)
        copies = _mk_copies(i, b)
        for cp in copies:
            cp.start()
        for cp in copies:
            cp.wait()
        return carry

    lax.fori_loop(0, _IPB, _row, 0)


def kernel(embeddings_table, length_q, length_k):
    del length_q, length_k
    run = functools.partial(
        pl.kernel,
        out_type=jax.ShapeDtypeStruct((_L, _D, _L), jnp.float32),
        mesh=plsc.VectorSubcoreMesh(core_axis_name="c", subcore_axis_name="s"),
        compiler_params=pltpu.CompilerParams(needs_layout_passes=False),
        scratch_types=[
            pltpu.VMEM((8 * 272,), jnp.float32),
            pltpu.VMEM((8 * 1024,), jnp.float32),
            pltpu.VMEM((8, 512), jnp.float32),
            pltpu.VMEM((2, 8, 512), jnp.float32),
            pltpu.SemaphoreType.DMA,
        ],
    )(_sc_body)
    out_t = run(jnp.pad(embeddings_table.T, ((0, 0), (0, 15))).reshape(-1))
    return jnp.swapaxes(out_t, 1, 2)


# speedup vs baseline: 3.6742x; 1.5462x over previous
import functools

import jax
import jax.numpy as jnp
from jax import lax
from jax.experimental import pallas as pl
from jax.experimental.pallas import tpu as pltpu
from jax.experimental.pallas import tpu_sc as plsc

_L = 2048
_D = 64
_IPB = _L // 4


def _mo16(x):
    return pl.multiple_of(x, 16)


def _sc_body(tab1_hbm, out_hbm, ttmp, band1d, cfill, staged, sem):
    nc = 2
    wid = lax.axis_index("s") * nc + lax.axis_index("c")
    octet = wid % 8
    ibase = (wid // 8) * _IPB
    d8 = octet * 8
    iota16 = lax.iota(jnp.int32, 16)

    pltpu.sync_copy(tab1_hbm.at[pl.ds(d8 * 272, 8 * 272)], ttmp)
    for dl in range(8):
        tbase = dl * 272
        t0 = jnp.full((16,), ttmp[pl.ds(tbase, 16)][0], jnp.float32)
        t256 = jnp.full((16,), ttmp[pl.ds(tbase + 256, 16)][0], jnp.float32)

        def _lo(m, carry, dl=dl, t0=t0):
            band1d[pl.ds(_mo16(dl * 1024 + 16 * m), 16)] = t0
            return carry

        def _mid(m, carry, dl=dl, tbase=tbase):
            vals = plsc.load_gather(ttmp, [tbase + 1 + 16 * m + iota16])
            band1d[pl.ds(_mo16(dl * 1024 + 384 + 16 * m), 16)] = vals
            return carry

        def _hi(m, carry, dl=dl, t256=t256):
            band1d[pl.ds(_mo16(dl * 1024 + 640 + 16 * m), 16)] = t256
            return carry

        def _cf(m, carry, dl=dl, t0=t0, t256=t256):
            cfill[dl, pl.ds(_mo16(16 * m), 16)] = t0
            cfill[dl, pl.ds(_mo16(256 + 16 * m), 16)] = t256
            return carry

        lax.fori_loop(0, 24, _lo, 0)
        lax.fori_loop(0, 16, _mid, 0)
        lax.fori_loop(0, 24, _hi, 0)
        lax.fori_loop(0, 16, _cf, 0)

    def _mk_copies(i, b):
        wstart = (jnp.clip(i - 128, 0, 1536) // 256) * 256
        nw = wstart // 256
        copies = [
            pltpu.make_async_copy(
                staged.at[b],
                out_hbm.at[i, pl.ds(d8, 8), pl.ds(wstart, 512)],
                sem,
            )
        ]
        for c in range(6):
            is_pre = c < nw
            dst_j = jnp.where(is_pre, 256 * c, wstart + 512 + 256 * c - 256 * nw)
            src_j = jnp.where(is_pre, 0, 256)
            copies.append(
                pltpu.make_async_copy(
                    cfill.at[:, pl.ds(src_j, 256)],
                    out_hbm.at[i, pl.ds(d8, 8), pl.ds(dst_j, 256)],
                    sem,
                )
            )
        return copies

    def _row(r, carry):
        i = ibase + r
        b = lax.rem(r, 2)
        wstart = (jnp.clip(i - 128, 0, 1536) // 256) * 256
        obase = wstart + 511 - i

        def _stage(m, carry):
            col = obase + 16 * m + iota16
            for dl in range(8):
                vals = plsc.load_gather(band1d, [col + 1024 * dl])
                staged[b, dl, pl.ds(_mo16(16 * m), 16)] = vals
            return carry

        lax.fori_loop(0, 32, _stage, 0)
        for cp in _mk_copies(i, b):
            cp.start()

        @pl.when(r > 0)
        def _drain_prev():
            for cp in _mk_copies(i - 1, lax.rem(r + 1, 2)):
                cp.wait()

        return carry

    lax.fori_loop(0, _IPB, _row, 0)
    for cp in _mk_copies(ibase + _IPB - 1, lax.rem(_IPB - 1, 2)):
        cp.wait()


def kernel(embeddings_table, length_q, length_k):
    del length_q, length_k
    run = functools.partial(
        pl.kernel,
        out_type=jax.ShapeDtypeStruct((_L, _D, _L), jnp.float32),
        mesh=plsc.VectorSubcoreMesh(core_axis_name="c", subcore_axis_name="s"),
        compiler_params=pltpu.CompilerParams(needs_layout_passes=False),
        scratch_types=[
            pltpu.VMEM((8 * 272,), jnp.float32),
            pltpu.VMEM((8 * 1024,), jnp.float32),
            pltpu.VMEM((8, 512), jnp.float32),
            pltpu.VMEM((2, 8, 512), jnp.float32),
            pltpu.SemaphoreType.DMA,
        ],
    )(_sc_body)
    out_t = run(jnp.pad(embeddings_table.T, ((0, 0), (0, 15))).reshape(-1))
    return jnp.swapaxes(out_t, 1, 2)


# speedup vs baseline: 3.6802x; 1.0016x over previous
import functools

import jax
import jax.numpy as jnp
from jax import lax
from jax.experimental import pallas as pl
from jax.experimental.pallas import tpu as pltpu
from jax.experimental.pallas import tpu_sc as plsc

_L = 2048
_D = 64
_IPB = _L // 4


def _mo16(x):
    return pl.multiple_of(x, 16)


def _sc_body(tab1_hbm, out_hbm, ttmp, band1d, cfill, staged, sem):
    nc = 2
    wid = lax.axis_index("s") * nc + lax.axis_index("c")
    octet = wid % 8
    ibase = (wid // 8) * _IPB
    d8 = octet * 8
    iota16 = lax.iota(jnp.int32, 16)

    pltpu.sync_copy(tab1_hbm.at[pl.ds(d8 * 272, 8 * 272)], ttmp)
    for dl in range(8):
        tbase = dl * 272
        t0 = jnp.full((16,), ttmp[pl.ds(tbase, 16)][0], jnp.float32)
        t256 = jnp.full((16,), ttmp[pl.ds(tbase + 256, 16)][0], jnp.float32)

        def _lo(m, carry, dl=dl, t0=t0):
            band1d[pl.ds(_mo16(dl * 1024 + 16 * m), 16)] = t0
            return carry

        def _mid(m, carry, dl=dl, tbase=tbase):
            vals = plsc.load_gather(ttmp, [tbase + 1 + 16 * m + iota16])
            band1d[pl.ds(_mo16(dl * 1024 + 384 + 16 * m), 16)] = vals
            return carry

        def _hi(m, carry, dl=dl, t256=t256):
            band1d[pl.ds(_mo16(dl * 1024 + 640 + 16 * m), 16)] = t256
            return carry

        def _cf(m, carry, dl=dl, t0=t0, t256=t256):
            cfill[dl, pl.ds(_mo16(16 * m), 16)] = t0
            cfill[dl, pl.ds(_mo16(256 + 16 * m), 16)] = t256
            return carry

        lax.fori_loop(0, 24, _lo, 0)
        lax.fori_loop(0, 16, _mid, 0)
        lax.fori_loop(0, 24, _hi, 0)
        lax.fori_loop(0, 16, _cf, 0)

    def _mk_copies(i, b):
        wstart = (jnp.clip(i - 128, 0, 1536) // 256) * 256
        nw = wstart // 256
        copies = [
            pltpu.make_async_copy(
                staged.at[b],
                out_hbm.at[i, pl.ds(d8, 8), pl.ds(wstart, 512)],
                sem,
            )
        ]
        for c in range(6):
            is_pre = c < nw
            dst_j = jnp.where(is_pre, 256 * c, wstart + 512 + 256 * c - 256 * nw)
            src_j = jnp.where(is_pre, 0, 256)
            copies.append(
                pltpu.make_async_copy(
                    cfill.at[:, pl.ds(src_j, 256)],
                    out_hbm.at[i, pl.ds(d8, 8), pl.ds(dst_j, 256)],
                    sem,
                )
            )
        return copies

    def _row(r, carry):
        i = ibase + r
        b = lax.rem(r, 3)
        wstart = (jnp.clip(i - 128, 0, 1536) // 256) * 256
        obase = wstart + 511 - i

        def _stage(m, carry):
            col = obase + 16 * m + iota16
            for dl in range(8):
                vals = plsc.load_gather(band1d, [col + 1024 * dl])
                staged[b, dl, pl.ds(_mo16(16 * m), 16)] = vals
            return carry

        lax.fori_loop(0, 32, _stage, 0)
        for cp in _mk_copies(i, b):
            cp.start()

        @pl.when(r > 1)
        def _drain_prev():
            for cp in _mk_copies(i - 2, lax.rem(r + 1, 3)):
                cp.wait()

        return carry

    lax.fori_loop(0, _IPB, _row, 0)
    for rr in (_IPB - 2, _IPB - 1):
        for cp in _mk_copies(ibase + rr, lax.rem(rr, 3)):
            cp.wait()


def kernel(embeddings_table, length_q, length_k):
    del length_q, length_k
    run = functools.partial(
        pl.kernel,
        out_type=jax.ShapeDtypeStruct((_L, _D, _L), jnp.float32),
        mesh=plsc.VectorSubcoreMesh(core_axis_name="c", subcore_axis_name="s"),
        compiler_params=pltpu.CompilerParams(needs_layout_passes=False),
        scratch_types=[
            pltpu.VMEM((8 * 272,), jnp.float32),
            pltpu.VMEM((8 * 1024,), jnp.float32),
            pltpu.VMEM((8, 512), jnp.float32),
            pltpu.VMEM((3, 8, 512), jnp.float32),
            pltpu.SemaphoreType.DMA,
        ],
    )(_sc_body)
    out_t = run(jnp.pad(embeddings_table.T, ((0, 0), (0, 15))).reshape(-1))
    return jnp.swapaxes(out_t, 1, 2)
